# baseline (speedup 1.0000x reference)
import functools

import jax
import jax.numpy as jnp
from jax import lax
from jax.experimental import pallas as pl
from jax.experimental.pallas import tpu as pltpu
from jax.experimental.pallas import tpu_sc as plsc

N_NODES = 10000
N_PAD = 10240
D = 128
DH = D // 2
N_EDGES = 320000

NC = 2
NS = 16
CHUNK = 256
CHUNKS_TOTAL = 1280
E_PAD = CHUNKS_TOTAL * CHUNK
CHUNKS_PER_TILE = CHUNKS_TOTAL // NS
STAGE = 8
ROWS_PER_TILE = N_PAD // NS

BLK = 1024


def _segsum_sc(x3, src2d, dst2d):
    mesh = plsc.VectorSubcoreMesh(core_axis_name="c", subcore_axis_name="s")

    @functools.partial(
        pl.kernel,
        mesh=mesh,
        compiler_params=pltpu.CompilerParams(use_tc_tiling_on_sc=False),
        out_type=jax.ShapeDtypeStruct((NC, N_PAD, DH), jnp.float32),
        scratch_types=[
            pltpu.VMEM((STAGE * CHUNK,), jnp.int32),
            pltpu.VMEM((STAGE * CHUNK,), jnp.int32),
            pltpu.VMEM((CHUNK, DH), jnp.float32),
            pltpu.VMEM((CHUNK, DH), jnp.float32),
            pltpu.VMEM_SHARED((N_PAD, DH), jnp.float32),
            pltpu.VMEM_SHARED((N_PAD, DH), jnp.float32),
            pltpu.SemaphoreType.DMA,
            pltpu.SemaphoreType.DMA,
        ],
    )
    def segsum(x_hbm, src_hbm, dst_hbm, out_hbm,
               src_v, dst_v, buf0, buf1, acc, x_s, sem0, sem1):
        c = lax.axis_index("c")
        s = lax.axis_index("s")

        row0 = s * ROWS_PER_TILE
        pltpu.sync_copy(x_hbm.at[c].at[pl.ds(row0, ROWS_PER_TILE)],
                        acc.at[pl.ds(row0, ROWS_PER_TILE)])
        pltpu.sync_copy(x_hbm.at[c].at[pl.ds(row0, ROWS_PER_TILE)],
                        x_s.at[pl.ds(row0, ROWS_PER_TILE)])

        plsc.subcore_barrier()

        for st in range(CHUNKS_PER_TILE // STAGE):
            base = (s * CHUNKS_PER_TILE + st * STAGE) * CHUNK
            pltpu.sync_copy(src_hbm.at[pl.ds(base, STAGE * CHUNK)], src_v)
            pltpu.sync_copy(dst_hbm.at[pl.ds(base, STAGE * CHUNK)], dst_v)

            pltpu.make_async_copy(x_s.at[src_v.at[pl.ds(0, CHUNK)]],
                                  buf0, sem0).start()

            def body(i, carry):
                j0 = 2 * i
                j1 = j0 + 1
                pltpu.make_async_copy(x_s.at[src_v.at[pl.ds(j0 * CHUNK, CHUNK)]], buf0,
                                      sem0).wait()
                pltpu.make_async_copy(x_s.at[src_v.at[pl.ds(j1 * CHUNK, CHUNK)]], buf1,
                                      sem1).start()
                pltpu.sync_copy(buf0, acc.at[dst_v.at[pl.ds(j0 * CHUNK, CHUNK)]], add=True)
                pltpu.make_async_copy(x_s.at[src_v.at[pl.ds(j1 * CHUNK, CHUNK)]], buf1,
                                      sem1).wait()

                @pl.when(j0 + 2 < STAGE)
                def _():
                    pltpu.make_async_copy(x_s.at[src_v.at[pl.ds((j0 + 2) * CHUNK, CHUNK)]],
                                          buf0, sem0).start()

                pltpu.sync_copy(buf1, acc.at[dst_v.at[pl.ds(j1 * CHUNK, CHUNK)]], add=True)
                return carry

            lax.fori_loop(0, STAGE // 2, body, 0)

        plsc.subcore_barrier()

        pltpu.sync_copy(acc.at[pl.ds(row0, ROWS_PER_TILE)],
                        out_hbm.at[c].at[pl.ds(row0, ROWS_PER_TILE)])

    return segsum(x3, src2d, dst2d)


def _elu(y):
    return jnp.where(y > 0, y, jnp.exp(jnp.minimum(y, 0.0)) - 1.0)


def _apply_tc_mid(p, Wt, b):
    def body(p_ref, w_ref, b_ref, o_ref):
        y = jnp.dot(p_ref[0], w_ref[:DH, :],
                    preferred_element_type=jnp.float32)
        y += jnp.dot(p_ref[1], w_ref[DH:, :],
                     preferred_element_type=jnp.float32)
        y = _elu(y + b_ref[...])
        o_ref[0] = y[:, :DH]
        o_ref[1] = y[:, DH:]

    return pl.pallas_call(
        body,
        grid=(N_PAD // BLK,),
        in_specs=[
            pl.BlockSpec((2, BLK, DH), lambda i: (0, i, 0)),
            pl.BlockSpec((D, D), lambda i: (0, 0)),
            pl.BlockSpec((1, D), lambda i: (0, 0)),
        ],
        out_specs=pl.BlockSpec((2, BLK, DH), lambda i: (0, i, 0)),
        out_shape=jax.ShapeDtypeStruct((NC, N_PAD, DH), jnp.float32),
    )(p, Wt, b)


def _apply_tc_final(p, Wt, b):
    blk = 1000

    def body(p_ref, w_ref, b_ref, o_ref):
        y = jnp.dot(p_ref[0], w_ref[:DH, :],
                    preferred_element_type=jnp.float32)
        y += jnp.dot(p_ref[1], w_ref[DH:, :],
                     preferred_element_type=jnp.float32)
        o_ref[...] = _elu(y + b_ref[...])

    return pl.pallas_call(
        body,
        grid=(N_NODES // blk,),
        in_specs=[
            pl.BlockSpec((2, blk, DH), lambda i: (0, i, 0)),
            pl.BlockSpec((D, D), lambda i: (0, 0)),
            pl.BlockSpec((1, D), lambda i: (0, 0)),
        ],
        out_specs=pl.BlockSpec((blk, D), lambda i: (i, 0)),
        out_shape=jax.ShapeDtypeStruct((N_NODES, D), jnp.float32),
    )(p, Wt, b)


def kernel(features, edge_index, order_attn, W1, b1, W2, b2):
    del order_attn
    pad = E_PAD - N_EDGES
    src1d = jnp.concatenate([edge_index[0], jnp.zeros((pad,), jnp.int32)])
    dst1d = jnp.concatenate(
        [edge_index[1], jnp.full((pad,), N_NODES, jnp.int32)])
    x3 = jnp.stack([features[:, :DH], features[:, DH:]])
    x3 = jnp.pad(x3, ((0, 0), (0, N_PAD - N_NODES), (0, 0)))

    p1 = _segsum_sc(x3, src1d, dst1d)
    h1 = _apply_tc_mid(p1, W1.T, b1.reshape(1, D))
    p2 = _segsum_sc(h1, src1d, dst1d)
    return _apply_tc_final(p2, W2.T, b2.reshape(1, D))


# speedup vs baseline: 1.3148x; 1.3148x over previous
import functools

import jax
import jax.numpy as jnp
from jax import lax
from jax.experimental import pallas as pl
from jax.experimental.pallas import tpu as pltpu
from jax.experimental.pallas import tpu_sc as plsc

N_NODES = 10000
N_PAD = 10240
D = 128
N_EDGES = 320000

NC = 2
NS = 16
CHUNK = 125
CHUNKS_PER_TILE = N_EDGES // (CHUNK * NC * NS)
HALF = CHUNKS_PER_TILE // 2
ROWS_PER_TILE = N_PAD // NS

BLK = 2048


def _segsum_sc(x, zeros, src2d, dst2d):
    x_rows = x.shape[0]
    mesh = plsc.VectorSubcoreMesh(core_axis_name="c", subcore_axis_name="s")

    @functools.partial(
        pl.kernel,
        mesh=mesh,
        out_type=jax.ShapeDtypeStruct((NC * N_PAD, D), jnp.float32),
        scratch_types=[
            pltpu.VMEM((HALF, CHUNK), jnp.int32),
            pltpu.VMEM((HALF, CHUNK), jnp.int32),
            pltpu.VMEM((CHUNK, D), jnp.float32),
            pltpu.VMEM((CHUNK, D), jnp.float32),
            pltpu.VMEM_SHARED((N_PAD, D), jnp.float32),
            pltpu.SemaphoreType.DMA,
            pltpu.SemaphoreType.DMA,
            pltpu.SemaphoreType.DMA,
            pltpu.SemaphoreType.DMA,
        ],
    )
    def segsum(x_hbm, z_hbm, src_hbm, dst_hbm, out_hbm,
               src_v, dst_v, buf0, buf1, acc, sem0, sem1, ssem0, ssem1):
        c = lax.axis_index("c")
        s = lax.axis_index("s")
        w = c * NS + s

        row0 = s * ROWS_PER_TILE

        if x_rows == N_PAD:
            @pl.when(c == 0)
            def _():
                pltpu.sync_copy(x_hbm.at[pl.ds(row0, ROWS_PER_TILE)],
                                acc.at[pl.ds(row0, ROWS_PER_TILE)])
        else:
            last = N_NODES - (NS - 1) * ROWS_PER_TILE

            @pl.when((c == 0) & (s < NS - 1))
            def _():
                pltpu.sync_copy(x_hbm.at[pl.ds(row0, ROWS_PER_TILE)],
                                acc.at[pl.ds(row0, ROWS_PER_TILE)])

            @pl.when((c == 0) & (s == NS - 1))
            def _():
                pltpu.sync_copy(x_hbm.at[pl.ds((NS - 1) * ROWS_PER_TILE,
                                               last)],
                                acc.at[pl.ds((NS - 1) * ROWS_PER_TILE,
                                             last)])

        @pl.when(c != 0)
        def _():
            pltpu.sync_copy(z_hbm.at[pl.ds(row0, ROWS_PER_TILE)],
                            acc.at[pl.ds(row0, ROWS_PER_TILE)])

        plsc.subcore_barrier()

        for h in range(2):
            base = w * CHUNKS_PER_TILE + h * HALF
            pltpu.sync_copy(src_hbm.at[pl.ds(base, HALF)], src_v)
            pltpu.sync_copy(dst_hbm.at[pl.ds(base, HALF)], dst_v)

            pltpu.make_async_copy(x_hbm.at[src_v.at[0]], buf0, sem0).start()

            def body(i, carry):
                j0 = 2 * i
                j1 = j0 + 1
                pltpu.make_async_copy(x_hbm.at[src_v.at[j0]], buf0,
                                      sem0).wait()
                pltpu.make_async_copy(x_hbm.at[src_v.at[j1]], buf1,
                                      sem1).start()
                pltpu.sync_copy(buf0, acc.at[dst_v.at[j0]], add=True)
                pltpu.make_async_copy(x_hbm.at[src_v.at[j1]], buf1,
                                      sem1).wait()

                @pl.when(j0 + 2 < HALF)
                def _():
                    pltpu.make_async_copy(x_hbm.at[src_v.at[j0 + 2]], buf0,
                                          sem0).start()

                pltpu.sync_copy(buf1, acc.at[dst_v.at[j1]], add=True)
                return carry

            lax.fori_loop(0, HALF // 2, body, 0)

        plsc.subcore_barrier()

        out_row = c * N_PAD + row0
        pltpu.sync_copy(acc.at[pl.ds(row0, ROWS_PER_TILE)],
                        out_hbm.at[pl.ds(out_row, ROWS_PER_TILE)])

    return segsum(x, zeros, src2d, dst2d)


def _apply_tc(p, Wt, b, out_rows, blk):
    def body(p_ref, w_ref, b_ref, o_ref):
        sblk = p_ref[0] + p_ref[1]
        y = jnp.dot(sblk, w_ref[...], preferred_element_type=jnp.float32)
        y = y + b_ref[...]
        o_ref[...] = jnp.where(y > 0, y, jnp.exp(jnp.minimum(y, 0.0)) - 1.0)

    return pl.pallas_call(
        body,
        grid=(out_rows // blk,),
        in_specs=[
            pl.BlockSpec((2, blk, D), lambda i: (0, i, 0)),
            pl.BlockSpec((D, D), lambda i: (0, 0)),
            pl.BlockSpec((1, D), lambda i: (0, 0)),
        ],
        out_specs=pl.BlockSpec((blk, D), lambda i: (i, 0)),
        out_shape=jax.ShapeDtypeStruct((out_rows, D), jnp.float32),
    )(p, Wt, b)


def kernel(features, edge_index, order_attn, W1, b1, W2, b2):
    del order_attn
    src2d = edge_index[0].reshape(NC * NS * CHUNKS_PER_TILE, CHUNK)
    dst2d = edge_index[1].reshape(NC * NS * CHUNKS_PER_TILE, CHUNK)
    zeros = jnp.zeros((N_PAD, D), jnp.float32)

    p1 = _segsum_sc(features, zeros, src2d, dst2d).reshape(NC, N_PAD, D)
    h1 = _apply_tc(p1, W1.T, b1.reshape(1, D), N_PAD, BLK)
    p2 = _segsum_sc(h1, zeros, src2d, dst2d).reshape(NC, N_PAD, D)
    return _apply_tc(p2, W2.T, b2.reshape(1, D), N_NODES, 1000)


# speedup vs baseline: 1.3270x; 1.0093x over previous
import functools

import jax
import jax.numpy as jnp
from jax import lax
from jax.experimental import pallas as pl
from jax.experimental.pallas import tpu as pltpu
from jax.experimental.pallas import tpu_sc as plsc

N_NODES = 10000
N_PAD = 10240
D = 128
N_EDGES = 320000

NC = 2
NS = 16
CHUNK = 125
CHUNKS_PER_TILE = N_EDGES // (CHUNK * NC * NS)
HALF = CHUNKS_PER_TILE // 2
ROWS_PER_TILE = N_PAD // NS

BLK = 2048


def _segsum_sc(x, zeros, src2d, dst2d):
    x_rows = x.shape[0]
    mesh = plsc.VectorSubcoreMesh(core_axis_name="c", subcore_axis_name="s")

    @functools.partial(
        pl.kernel,
        mesh=mesh,
        out_type=jax.ShapeDtypeStruct((NC * N_PAD, D), jnp.float32),
        scratch_types=[
            pltpu.VMEM((HALF, CHUNK), jnp.int32),
            pltpu.VMEM((HALF, CHUNK), jnp.int32),
            pltpu.VMEM((CHUNK, D), jnp.float32),
            pltpu.VMEM((CHUNK, D), jnp.float32),
            pltpu.VMEM_SHARED((N_PAD, D), jnp.float32),
            pltpu.SemaphoreType.DMA,
            pltpu.SemaphoreType.DMA,
            pltpu.SemaphoreType.DMA,
            pltpu.SemaphoreType.DMA,
        ],
    )
    def segsum(x_hbm, z_hbm, src_hbm, dst_hbm, out_hbm,
               src_v, dst_v, buf0, buf1, acc, sem0, sem1, ssem0, ssem1):
        c = lax.axis_index("c")
        s = lax.axis_index("s")
        w = c * NS + s

        row0 = s * ROWS_PER_TILE

        if x_rows == N_PAD:
            @pl.when(c == 0)
            def _():
                pltpu.sync_copy(x_hbm.at[pl.ds(row0, ROWS_PER_TILE)],
                                acc.at[pl.ds(row0, ROWS_PER_TILE)])
        else:
            last = N_NODES - (NS - 1) * ROWS_PER_TILE

            @pl.when((c == 0) & (s < NS - 1))
            def _():
                pltpu.sync_copy(x_hbm.at[pl.ds(row0, ROWS_PER_TILE)],
                                acc.at[pl.ds(row0, ROWS_PER_TILE)])

            @pl.when((c == 0) & (s == NS - 1))
            def _():
                pltpu.sync_copy(x_hbm.at[pl.ds((NS - 1) * ROWS_PER_TILE,
                                               last)],
                                acc.at[pl.ds((NS - 1) * ROWS_PER_TILE,
                                             last)])

        @pl.when(c != 0)
        def _():
            pltpu.sync_copy(z_hbm.at[pl.ds(row0, ROWS_PER_TILE)],
                            acc.at[pl.ds(row0, ROWS_PER_TILE)])

        pltpu.sync_copy(src_hbm.at[pl.ds(w * CHUNKS_PER_TILE, HALF)], src_v)
        pltpu.sync_copy(dst_hbm.at[pl.ds(w * CHUNKS_PER_TILE, HALF)], dst_v)
        pltpu.make_async_copy(x_hbm.at[src_v.at[0]], buf0, sem0).start()

        plsc.subcore_barrier()

        for h in range(2):
            if h:
                base = w * CHUNKS_PER_TILE + h * HALF
                pltpu.sync_copy(src_hbm.at[pl.ds(base, HALF)], src_v)
                pltpu.sync_copy(dst_hbm.at[pl.ds(base, HALF)], dst_v)
                pltpu.make_async_copy(x_hbm.at[src_v.at[0]], buf0,
                                      sem0).start()

            def body(i, carry):
                j0 = 2 * i
                j1 = j0 + 1
                pltpu.make_async_copy(x_hbm.at[src_v.at[j0]], buf0,
                                      sem0).wait()
                pltpu.make_async_copy(x_hbm.at[src_v.at[j1]], buf1,
                                      sem1).start()
                pltpu.sync_copy(buf0, acc.at[dst_v.at[j0]], add=True)
                pltpu.make_async_copy(x_hbm.at[src_v.at[j1]], buf1,
                                      sem1).wait()

                @pl.when(j0 + 2 < HALF)
                def _():
                    pltpu.make_async_copy(x_hbm.at[src_v.at[j0 + 2]], buf0,
                                          sem0).start()

                pltpu.sync_copy(buf1, acc.at[dst_v.at[j1]], add=True)
                return carry

            lax.fori_loop(0, HALF // 2, body, 0)

        plsc.subcore_barrier()

        out_row = c * N_PAD + row0
        pltpu.sync_copy(acc.at[pl.ds(row0, ROWS_PER_TILE)],
                        out_hbm.at[pl.ds(out_row, ROWS_PER_TILE)])

    return segsum(x, zeros, src2d, dst2d)


def _apply_tc(p, Wt, b, out_rows, blk):
    def body(p_ref, w_ref, b_ref, o_ref):
        sblk = p_ref[0] + p_ref[1]
        y = jnp.dot(sblk, w_ref[...], preferred_element_type=jnp.float32)
        y = y + b_ref[...]
        o_ref[...] = jnp.where(y > 0, y, jnp.exp(jnp.minimum(y, 0.0)) - 1.0)

    return pl.pallas_call(
        body,
        grid=(out_rows // blk,),
        in_specs=[
            pl.BlockSpec((2, blk, D), lambda i: (0, i, 0)),
            pl.BlockSpec((D, D), lambda i: (0, 0)),
            pl.BlockSpec((1, D), lambda i: (0, 0)),
        ],
        out_specs=pl.BlockSpec((blk, D), lambda i: (i, 0)),
        out_shape=jax.ShapeDtypeStruct((out_rows, D), jnp.float32),
    )(p, Wt, b)


def kernel(features, edge_index, order_attn, W1, b1, W2, b2):
    del order_attn
    src2d = edge_index[0].reshape(NC * NS * CHUNKS_PER_TILE, CHUNK)
    dst2d = edge_index[1].reshape(NC * NS * CHUNKS_PER_TILE, CHUNK)
    zeros = jnp.zeros((N_PAD, D), jnp.float32)

    p1 = _segsum_sc(features, zeros, src2d, dst2d).reshape(NC, N_PAD, D)
    h1 = _apply_tc(p1, W1.T, b1.reshape(1, D), N_PAD, BLK)
    p2 = _segsum_sc(h1, zeros, src2d, dst2d).reshape(NC, N_PAD, D)
    return _apply_tc(p2, W2.T, b2.reshape(1, D), N_NODES, 1000)
